# baseline (speedup 1.0000x reference)
import functools

import jax
import jax.numpy as jnp
from jax import lax
from jax.experimental import pallas as pl
from jax.experimental.pallas import tpu as pltpu
from jax.experimental.pallas import tpu_sc as plsc

NC = 2
NS = 16
NW = NC * NS
CW = 16


def _silu(x):
    return x * (1.0 / (1.0 + jnp.exp(-x)))


def _sc_mesh():
    return plsc.VectorSubcoreMesh(
        core_axis_name="c", subcore_axis_name="s", num_cores=NC, num_subcores=NS
    )


def _sc_gather_pair(table, row, col, tc_tiling, ch):
    E = row.shape[0]
    N, W = table.shape
    EW = E // NW
    NCHK = EW // ch

    @functools.partial(
        pl.kernel,
        out_type=(
            jax.ShapeDtypeStruct((E, W), jnp.float32),
            jax.ShapeDtypeStruct((E, W), jnp.float32),
        ),
        mesh=_sc_mesh(),
        scratch_types=[
            pltpu.VMEM((2, ch), jnp.int32),
            pltpu.VMEM((2, ch), jnp.int32),
            pltpu.VMEM((2, ch, W), jnp.float32),
            pltpu.VMEM((2, ch, W), jnp.float32),
            pltpu.SemaphoreType.DMA((2,)),
        ],
        compiler_params=pltpu.CompilerParams(use_tc_tiling_on_sc=tc_tiling),
    )
    def k(t_hbm, row_hbm, col_hbm, orow_hbm, ocol_hbm, idxr, idxc, bufr, bufc,
          gsem):
        wid = lax.axis_index("s") * NC + lax.axis_index("c")
        wbase = wid * EW

        def gdescs(par):
            return (
                pltpu.make_async_copy(t_hbm.at[idxr.at[par]], bufr.at[par],
                                      gsem.at[par]),
                pltpu.make_async_copy(t_hbm.at[idxc.at[par]], bufc.at[par],
                                      gsem.at[par]),
            )

        def start_chunk(c, par):
            base = wbase + c * ch
            pltpu.sync_copy(row_hbm.at[pl.ds(base, ch)], idxr.at[par])
            pltpu.sync_copy(col_hbm.at[pl.ds(base, ch)], idxc.at[par])
            for d in gdescs(par):
                d.start()

        start_chunk(0, 0)

        def body(c, carry):
            par = lax.rem(c, 2)
            npar = 1 - par

            @pl.when(c < NCHK - 1)
            def _():
                start_chunk(c + 1, npar)

            for d in gdescs(par):
                d.wait()
            base = wbase + c * ch
            pltpu.sync_copy(bufr.at[par], orow_hbm.at[pl.ds(base, ch)])
            pltpu.sync_copy(bufc.at[par], ocol_hbm.at[pl.ds(base, ch)])
            return carry

        lax.fori_loop(0, NCHK, body, 0)

    return k(table, row, col)


def _sc_scatter_one(vals, row, zeros_init, tc_tiling, ch):
    E = row.shape[0]
    N, W = zeros_init.shape
    EW = E // NW
    NCHK = EW // ch
    RT = N // NS

    @functools.partial(
        pl.kernel,
        out_type=jax.ShapeDtypeStruct((NC * N, W), jnp.float32),
        mesh=_sc_mesh(),
        scratch_types=[
            pltpu.VMEM((2, ch), jnp.int32),
            pltpu.VMEM((2, ch, W), jnp.float32),
            pltpu.SemaphoreType.DMA((2,)),
            pltpu.VMEM_SHARED((N, W), jnp.float32),
        ],
        compiler_params=pltpu.CompilerParams(use_tc_tiling_on_sc=tc_tiling),
    )
    def k(v_hbm, row_hbm, z_hbm, out_hbm, idx, buf, lsem, acc):
        cid = lax.axis_index("c")
        sid = lax.axis_index("s")
        wid = sid * NC + cid
        wbase = wid * EW

        def ldescs(c, par):
            base = wbase + c * ch
            return (
                pltpu.make_async_copy(row_hbm.at[pl.ds(base, ch)],
                                      idx.at[par], lsem.at[par]),
                pltpu.make_async_copy(v_hbm.at[pl.ds(base, ch)],
                                      buf.at[par], lsem.at[par]),
            )

        for d in ldescs(0, 0):
            d.start()
        pltpu.sync_copy(z_hbm.at[pl.ds(sid * RT, RT)], acc.at[pl.ds(sid * RT, RT)])
        plsc.subcore_barrier()

        def body(c, carry):
            par = lax.rem(c, 2)
            npar = 1 - par

            @pl.when(c < NCHK - 1)
            def _():
                for d in ldescs(c + 1, npar):
                    d.start()

            for d in ldescs(c, par):
                d.wait()
            pltpu.sync_copy(buf.at[par], acc.at[idx.at[par]], add=True)
            return carry

        lax.fori_loop(0, NCHK, body, 0)
        plsc.subcore_barrier()
        pltpu.sync_copy(
            acc.at[pl.ds(sid * RT, RT)],
            out_hbm.at[pl.ds(cid * N + sid * RT, RT)],
        )

    return k(vals, row, zeros_init)


def _tc_edge(hrow, hcol, crow, ccol, Wa, Wb, wr, be1, We2, be2, Wc1, bc1, wc2):
    E, D = hrow.shape
    BE = 1600

    def body(hr_ref, hc_ref, cr_ref, cc_ref, Wa_ref, Wb_ref, wr_ref, be1_ref,
             We2_ref, be2_ref, Wc1_ref, bc1_ref, wc2_ref, om_ref, oc_ref):
        bf = jnp.bfloat16
        cd = cr_ref[...] - cc_ref[...]
        radial = jnp.sum(cd * cd, axis=-1, keepdims=True)
        t1 = _silu(
            jnp.dot(hr_ref[...].astype(bf), Wa_ref[...].astype(bf),
                    preferred_element_type=jnp.float32)
            + jnp.dot(hc_ref[...].astype(bf), Wb_ref[...].astype(bf),
                      preferred_element_type=jnp.float32)
            + radial * wr_ref[...]
            + be1_ref[...]
        )
        msg = _silu(
            jnp.dot(t1.astype(bf), We2_ref[...].astype(bf),
                    preferred_element_type=jnp.float32)
            + be2_ref[...]
        )
        t3 = _silu(
            jnp.dot(msg.astype(bf), Wc1_ref[...].astype(bf),
                    preferred_element_type=jnp.float32)
            + bc1_ref[...]
        )
        cw = jnp.sum(t3 * wc2_ref[...], axis=-1, keepdims=True)
        om_ref[...] = msg
        oc_ref[...] = cd * cw

    wspec = pl.BlockSpec((128, 128), lambda i: (0, 0))
    vspec = pl.BlockSpec((1, 128), lambda i: (0, 0))
    espec = pl.BlockSpec((BE, D), lambda i: (i, 0))
    cspec = pl.BlockSpec((BE, CW), lambda i: (i, 0))
    return pl.pallas_call(
        body,
        grid=(E // BE,),
        in_specs=[
            espec, espec, cspec, cspec,
            wspec, wspec, vspec, vspec, wspec, vspec, wspec, vspec, vspec,
        ],
        out_specs=[espec, cspec],
        out_shape=[
            jax.ShapeDtypeStruct((E, D), jnp.float32),
            jax.ShapeDtypeStruct((E, CW), jnp.float32),
        ],
        compiler_params=pltpu.CompilerParams(
            dimension_semantics=("arbitrary",)
        ),
    )(hrow, hcol, crow, ccol, Wa, Wb, wr, be1, We2, be2, Wc1, bc1, wc2)


def _tc_node(h, cpad, phs, pcs, Wn1a, Wn1b, bn1, Wn2, bn2):
    N, D = h.shape
    BN = 2000
    nph = len(phs)
    npc = len(pcs)

    def body(*refs):
        h_ref = refs[0]
        cp_ref = refs[1]
        ph_refs = refs[2:2 + nph]
        pc_refs = refs[2 + nph:2 + nph + npc]
        Wa_ref, Wb_ref, b1_ref, W2_ref, b2_ref, oh_ref, oc_ref = \
            refs[2 + nph + npc:]
        bf = jnp.bfloat16
        hh = h_ref[...]
        agg = ph_refs[0][...]
        for r in ph_refs[1:]:
            agg = agg + r[...]
        aggc = pc_refs[0][...]
        for r in pc_refs[1:]:
            aggc = aggc + r[...]
        t = _silu(
            jnp.dot(hh.astype(bf), Wa_ref[...].astype(bf),
                    preferred_element_type=jnp.float32)
            + jnp.dot(agg.astype(bf), Wb_ref[...].astype(bf),
                      preferred_element_type=jnp.float32)
            + b1_ref[...]
        )
        oh_ref[...] = (
            jnp.dot(t.astype(bf), W2_ref[...].astype(bf),
                    preferred_element_type=jnp.float32)
            + b2_ref[...]
            + hh
        )
        oc_ref[...] = cp_ref[...] + aggc

    wspec = pl.BlockSpec((128, 128), lambda i: (0, 0))
    vspec = pl.BlockSpec((1, 128), lambda i: (0, 0))
    nspec = pl.BlockSpec((BN, D), lambda i: (i, 0))
    cspec = pl.BlockSpec((BN, CW), lambda i: (i, 0))
    return pl.pallas_call(
        body,
        grid=(N // BN,),
        in_specs=[nspec, cspec] + [nspec] * nph + [cspec] * npc
                 + [wspec, wspec, vspec, wspec, vspec],
        out_specs=[nspec, cspec],
        out_shape=[
            jax.ShapeDtypeStruct((N, D), jnp.float32),
            jax.ShapeDtypeStruct((N, CW), jnp.float32),
        ],
        compiler_params=pltpu.CompilerParams(
            dimension_semantics=("arbitrary",)
        ),
    )(h, cpad, *phs, *pcs, Wn1a, Wn1b, bn1, Wn2, bn2)


def kernel(h, coords, edge_index, We1, be1, We2, be2, Wn1, bn1, Wn2, bn2, Wc1,
           bc1, Wc2):
    N, D = h.shape
    row = edge_index[0].astype(jnp.int32)
    col = edge_index[1].astype(jnp.int32)
    cpad = jnp.pad(
        coords.astype(jnp.float32), ((0, 0), (0, CW - coords.shape[1]))
    )
    E = row.shape[0]
    E2 = E // 2
    CHH = 40

    Wa = We1[0:D]
    Wb = We1[D:2 * D]
    wr = We1[2 * D].reshape(1, D)
    ew = (Wa, Wb, wr, be1.reshape(1, D), We2, be2.reshape(1, D), Wc1,
          bc1.reshape(1, D), Wc2.reshape(1, D))

    NP = ((N + 8 * NS - 1) // (8 * NS)) * (8 * NS)
    zh = jnp.zeros((NP, D), jnp.float32)
    zc = jnp.zeros((NP, CW), jnp.float32)

    rowA, rowB = row[0:E2], row[E2:E]
    colA, colB = col[0:E2], col[E2:E]

    hrA, hcA = _sc_gather_pair(h, rowA, colA, tc_tiling=True, ch=CHH)
    crA, ccA = _sc_gather_pair(cpad, rowA, colA, tc_tiling=False, ch=CHH)
    hrB, hcB = _sc_gather_pair(h, rowB, colB, tc_tiling=True, ch=CHH)
    crB, ccB = _sc_gather_pair(cpad, rowB, colB, tc_tiling=False, ch=CHH)
    msgA, cdwA = _tc_edge(hrA, hcA, crA, ccA, *ew)
    phA = _sc_scatter_one(msgA, rowA, zh, tc_tiling=True, ch=CHH)
    pcA = _sc_scatter_one(cdwA, rowA, zc, tc_tiling=False, ch=CHH)
    msgB, cdwB = _tc_edge(hrB, hcB, crB, ccB, *ew)
    phB = _sc_scatter_one(msgB, rowB, zh, tc_tiling=True, ch=CHH)
    pcB = _sc_scatter_one(cdwB, rowB, zc, tc_tiling=False, ch=CHH)

    h_new, coords_new_pad = _tc_node(
        h, cpad,
        (phA[0:N], phA[NP:NP + N], phB[0:N], phB[NP:NP + N]),
        (pcA[0:N], pcA[NP:NP + N], pcB[0:N], pcB[NP:NP + N]),
        Wn1[0:D], Wn1[D:2 * D], bn1.reshape(1, D), Wn2, bn2.reshape(1, D),
    )
    return h_new, coords_new_pad[:, 0:coords.shape[1]]


# speedup vs baseline: 1.0862x; 1.0862x over previous
import functools

import jax
import jax.numpy as jnp
from jax import lax
from jax.experimental import pallas as pl
from jax.experimental.pallas import tpu as pltpu
from jax.experimental.pallas import tpu_sc as plsc

NC = 2
NS = 16
NW = NC * NS
CH = 80
CW = 16


def _silu(x):
    return x * (1.0 / (1.0 + jnp.exp(-x)))


def _sc_mesh():
    return plsc.VectorSubcoreMesh(
        core_axis_name="c", subcore_axis_name="s", num_cores=NC, num_subcores=NS
    )


def _sc_gather(h, cpad, row, col):
    E = row.shape[0]
    N, D = h.shape
    EW = E // NW
    NCHK = EW // CH

    @functools.partial(
        pl.kernel,
        out_type=(
            jax.ShapeDtypeStruct((E, D), jnp.float32),
            jax.ShapeDtypeStruct((E, D), jnp.float32),
            jax.ShapeDtypeStruct((E, CW), jnp.float32),
            jax.ShapeDtypeStruct((E, CW), jnp.float32),
        ),
        mesh=_sc_mesh(),
        scratch_types=[
            pltpu.VMEM((2, CH), jnp.int32),
            pltpu.VMEM((2, CH), jnp.int32),
            pltpu.VMEM((2, CH, D), jnp.float32),
            pltpu.VMEM((2, CH, D), jnp.float32),
            pltpu.VMEM((2, CH, CW), jnp.float32),
            pltpu.VMEM((2, CH, CW), jnp.float32),
            pltpu.SemaphoreType.DMA((2,)),
        ],
        compiler_params=pltpu.CompilerParams(use_tc_tiling_on_sc=False),
    )
    def k(h_hbm, c_hbm, row_hbm, col_hbm, ohr_hbm, ohc_hbm, ocr_hbm, occ_hbm,
          idxr, idxc, bufhr, bufhc, bufcr, bufcc, gsem):
        wid = lax.axis_index("s") * NC + lax.axis_index("c")
        wbase = wid * EW

        def gdescs(par):
            return (
                pltpu.make_async_copy(h_hbm.at[idxr.at[par]], bufhr.at[par],
                                      gsem.at[par]),
                pltpu.make_async_copy(h_hbm.at[idxc.at[par]], bufhc.at[par],
                                      gsem.at[par]),
                pltpu.make_async_copy(c_hbm.at[idxr.at[par]], bufcr.at[par],
                                      gsem.at[par]),
                pltpu.make_async_copy(c_hbm.at[idxc.at[par]], bufcc.at[par],
                                      gsem.at[par]),
            )

        def start_chunk(c, par):
            base = wbase + c * CH
            pltpu.sync_copy(row_hbm.at[pl.ds(base, CH)], idxr.at[par])
            pltpu.sync_copy(col_hbm.at[pl.ds(base, CH)], idxc.at[par])
            for d in gdescs(par):
                d.start()

        start_chunk(0, 0)

        def body(c, carry):
            par = lax.rem(c, 2)
            npar = 1 - par

            @pl.when(c < NCHK - 1)
            def _():
                start_chunk(c + 1, npar)

            for d in gdescs(par):
                d.wait()
            base = wbase + c * CH
            pltpu.sync_copy(bufhr.at[par], ohr_hbm.at[pl.ds(base, CH)])
            pltpu.sync_copy(bufhc.at[par], ohc_hbm.at[pl.ds(base, CH)])
            pltpu.sync_copy(bufcr.at[par], ocr_hbm.at[pl.ds(base, CH)])
            pltpu.sync_copy(bufcc.at[par], occ_hbm.at[pl.ds(base, CH)])
            return carry

        lax.fori_loop(0, NCHK, body, 0)

    return k(h, cpad, row, col)


def _sc_scatter(msg, cdw, row, zh, zc):
    E = row.shape[0]
    N, D = zh.shape
    EW = E // NW
    NCHK = EW // CH
    RT = N // NS

    @functools.partial(
        pl.kernel,
        out_type=(
            jax.ShapeDtypeStruct((NC * N, D), jnp.float32),
            jax.ShapeDtypeStruct((NC * N, CW), jnp.float32),
        ),
        mesh=_sc_mesh(),
        scratch_types=[
            pltpu.VMEM((2, CH), jnp.int32),
            pltpu.VMEM((2, CH, D), jnp.float32),
            pltpu.VMEM((2, CH, CW), jnp.float32),
            pltpu.SemaphoreType.DMA((2,)),
            pltpu.VMEM_SHARED((N, D), jnp.float32),
            pltpu.VMEM_SHARED((N, CW), jnp.float32),
        ],
        compiler_params=pltpu.CompilerParams(use_tc_tiling_on_sc=False),
    )
    def k(msg_hbm, cdw_hbm, row_hbm, zh_hbm, zc_hbm, oh_hbm, oc_hbm,
          idx, bufh, bufc, lsem, acch, accc):
        cid = lax.axis_index("c")
        sid = lax.axis_index("s")
        wid = sid * NC + cid
        wbase = wid * EW

        def ldescs(c, par):
            base = wbase + c * CH
            return (
                pltpu.make_async_copy(row_hbm.at[pl.ds(base, CH)],
                                      idx.at[par], lsem.at[par]),
                pltpu.make_async_copy(msg_hbm.at[pl.ds(base, CH)],
                                      bufh.at[par], lsem.at[par]),
                pltpu.make_async_copy(cdw_hbm.at[pl.ds(base, CH)],
                                      bufc.at[par], lsem.at[par]),
            )

        for d in ldescs(0, 0):
            d.start()
        pltpu.sync_copy(zh_hbm.at[pl.ds(sid * RT, RT)], acch.at[pl.ds(sid * RT, RT)])
        pltpu.sync_copy(zc_hbm.at[pl.ds(sid * RT, RT)], accc.at[pl.ds(sid * RT, RT)])
        plsc.subcore_barrier()

        def body(c, carry):
            par = lax.rem(c, 2)
            npar = 1 - par

            @pl.when(c < NCHK - 1)
            def _():
                for d in ldescs(c + 1, npar):
                    d.start()

            for d in ldescs(c, par):
                d.wait()
            pltpu.sync_copy(bufh.at[par], acch.at[idx.at[par]], add=True)
            pltpu.sync_copy(bufc.at[par], accc.at[idx.at[par]], add=True)
            return carry

        lax.fori_loop(0, NCHK, body, 0)
        plsc.subcore_barrier()
        pltpu.sync_copy(
            acch.at[pl.ds(sid * RT, RT)],
            oh_hbm.at[pl.ds(cid * N + sid * RT, RT)],
        )
        pltpu.sync_copy(
            accc.at[pl.ds(sid * RT, RT)],
            oc_hbm.at[pl.ds(cid * N + sid * RT, RT)],
        )

    return k(msg, cdw, row, zh, zc)


def _tc_edge(hrow, hcol, crow, ccol, Wa, Wb, wr, be1, We2, be2, Wc1, bc1, wc2):
    E, D = hrow.shape
    BE = 1600

    def body(hr_ref, hc_ref, cr_ref, cc_ref, Wa_ref, Wb_ref, wr_ref, be1_ref,
             We2_ref, be2_ref, Wc1_ref, bc1_ref, wc2_ref, om_ref, oc_ref):
        bf = jnp.bfloat16
        cd = cr_ref[...] - cc_ref[...]
        radial = jnp.sum(cd * cd, axis=-1, keepdims=True)
        t1 = _silu(
            jnp.dot(hr_ref[...].astype(bf), Wa_ref[...].astype(bf),
                    preferred_element_type=jnp.float32)
            + jnp.dot(hc_ref[...].astype(bf), Wb_ref[...].astype(bf),
                      preferred_element_type=jnp.float32)
            + radial * wr_ref[...]
            + be1_ref[...]
        )
        msg = _silu(
            jnp.dot(t1.astype(bf), We2_ref[...].astype(bf),
                    preferred_element_type=jnp.float32)
            + be2_ref[...]
        )
        t3 = _silu(
            jnp.dot(msg.astype(bf), Wc1_ref[...].astype(bf),
                    preferred_element_type=jnp.float32)
            + bc1_ref[...]
        )
        cw = jnp.sum(t3 * wc2_ref[...], axis=-1, keepdims=True)
        om_ref[...] = msg
        oc_ref[...] = cd * cw

    wspec = pl.BlockSpec((128, 128), lambda i: (0, 0))
    vspec = pl.BlockSpec((1, 128), lambda i: (0, 0))
    espec = pl.BlockSpec((BE, D), lambda i: (i, 0))
    cspec = pl.BlockSpec((BE, CW), lambda i: (i, 0))
    return pl.pallas_call(
        body,
        grid=(E // BE,),
        in_specs=[
            espec, espec, cspec, cspec,
            wspec, wspec, vspec, vspec, wspec, vspec, wspec, vspec, vspec,
        ],
        out_specs=[espec, cspec],
        out_shape=[
            jax.ShapeDtypeStruct((E, D), jnp.float32),
            jax.ShapeDtypeStruct((E, CW), jnp.float32),
        ],
        compiler_params=pltpu.CompilerParams(
            dimension_semantics=("arbitrary",)
        ),
    )(hrow, hcol, crow, ccol, Wa, Wb, wr, be1, We2, be2, Wc1, bc1, wc2)


def _tc_node(h, cpad, p0h, p1h, p0c, p1c, Wn1a, Wn1b, bn1, Wn2, bn2):
    N, D = h.shape
    BN = 2000

    def body(h_ref, cp_ref, p0h_ref, p1h_ref, p0c_ref, p1c_ref, Wa_ref,
             Wb_ref, b1_ref, W2_ref, b2_ref, oh_ref, oc_ref):
        bf = jnp.bfloat16
        hh = h_ref[...]
        agg = p0h_ref[...] + p1h_ref[...]
        t = _silu(
            jnp.dot(hh.astype(bf), Wa_ref[...].astype(bf),
                    preferred_element_type=jnp.float32)
            + jnp.dot(agg.astype(bf), Wb_ref[...].astype(bf),
                      preferred_element_type=jnp.float32)
            + b1_ref[...]
        )
        oh_ref[...] = (
            jnp.dot(t.astype(bf), W2_ref[...].astype(bf),
                    preferred_element_type=jnp.float32)
            + b2_ref[...]
            + hh
        )
        oc_ref[...] = cp_ref[...] + p0c_ref[...] + p1c_ref[...]

    wspec = pl.BlockSpec((128, 128), lambda i: (0, 0))
    vspec = pl.BlockSpec((1, 128), lambda i: (0, 0))
    nspec = pl.BlockSpec((BN, D), lambda i: (i, 0))
    cspec = pl.BlockSpec((BN, CW), lambda i: (i, 0))
    return pl.pallas_call(
        body,
        grid=(N // BN,),
        in_specs=[nspec, cspec, nspec, nspec, cspec, cspec,
                  wspec, wspec, vspec, wspec, vspec],
        out_specs=[nspec, cspec],
        out_shape=[
            jax.ShapeDtypeStruct((N, D), jnp.float32),
            jax.ShapeDtypeStruct((N, CW), jnp.float32),
        ],
        compiler_params=pltpu.CompilerParams(
            dimension_semantics=("arbitrary",)
        ),
    )(h, cpad, p0h, p1h, p0c, p1c, Wn1a, Wn1b, bn1, Wn2, bn2)


def kernel(h, coords, edge_index, We1, be1, We2, be2, Wn1, bn1, Wn2, bn2, Wc1,
           bc1, Wc2):
    N, D = h.shape
    row = edge_index[0].astype(jnp.int32)
    col = edge_index[1].astype(jnp.int32)
    cpad = jnp.pad(
        coords.astype(jnp.float32), ((0, 0), (0, CW - coords.shape[1]))
    )

    hrow, hcol, crow, ccol = _sc_gather(h, cpad, row, col)

    Wa = We1[0:D]
    Wb = We1[D:2 * D]
    wr = We1[2 * D].reshape(1, D)
    msg, cdw = _tc_edge(
        hrow, hcol, crow, ccol, Wa, Wb, wr, be1.reshape(1, D), We2,
        be2.reshape(1, D), Wc1, bc1.reshape(1, D), Wc2.reshape(1, D),
    )

    zh = jnp.zeros((N, D), jnp.float32)
    zc = jnp.zeros((N, CW), jnp.float32)
    ph, pc = _sc_scatter(msg, cdw, row, zh, zc)

    h_new, coords_new_pad = _tc_node(
        h, cpad, ph[0:N], ph[N:2 * N], pc[0:N], pc[N:2 * N],
        Wn1[0:D], Wn1[D:2 * D], bn1.reshape(1, D), Wn2, bn2.reshape(1, D),
    )
    return h_new, coords_new_pad[:, 0:coords.shape[1]]


# speedup vs baseline: 1.0927x; 1.0060x over previous
import functools

import jax
import jax.numpy as jnp
from jax import lax
from jax.experimental import pallas as pl
from jax.experimental.pallas import tpu as pltpu
from jax.experimental.pallas import tpu_sc as plsc

NC = 2
NS = 16
NW = NC * NS
CH = 80
CW = 16


def _silu(x):
    return x * (1.0 / (1.0 + jnp.exp(-x)))


def _sc_mesh():
    return plsc.VectorSubcoreMesh(
        core_axis_name="c", subcore_axis_name="s", num_cores=NC, num_subcores=NS
    )


def _sc_gather(h, cpad, row, col):
    E = row.shape[0]
    N, D = h.shape
    EW = E // NW
    NCHK = EW // CH

    @functools.partial(
        pl.kernel,
        out_type=(
            jax.ShapeDtypeStruct((E, D), jnp.float32),
            jax.ShapeDtypeStruct((E, D), jnp.float32),
            jax.ShapeDtypeStruct((E, CW), jnp.float32),
            jax.ShapeDtypeStruct((E, CW), jnp.float32),
        ),
        mesh=_sc_mesh(),
        scratch_types=[
            pltpu.VMEM((2, CH), jnp.int32),
            pltpu.VMEM((2, CH), jnp.int32),
            pltpu.VMEM((2, CH, D), jnp.float32),
            pltpu.VMEM((2, CH, D), jnp.float32),
            pltpu.VMEM((2, CH, CW), jnp.float32),
            pltpu.VMEM((2, CH, CW), jnp.float32),
            pltpu.SemaphoreType.DMA((2,)),
            pltpu.SemaphoreType.DMA((2,)),
        ],
        compiler_params=pltpu.CompilerParams(use_tc_tiling_on_sc=False),
    )
    def k(h_hbm, c_hbm, row_hbm, col_hbm, ohr_hbm, ohc_hbm, ocr_hbm, occ_hbm,
          idxr, idxc, bufhr, bufhc, bufcr, bufcc, gsem, wsem):
        wid = lax.axis_index("s") * NC + lax.axis_index("c")
        wbase = wid * EW

        def wdescs(c, par):
            base = wbase + c * CH
            return (
                pltpu.make_async_copy(bufhr.at[par],
                                      ohr_hbm.at[pl.ds(base, CH)],
                                      wsem.at[par]),
                pltpu.make_async_copy(bufhc.at[par],
                                      ohc_hbm.at[pl.ds(base, CH)],
                                      wsem.at[par]),
                pltpu.make_async_copy(bufcr.at[par],
                                      ocr_hbm.at[pl.ds(base, CH)],
                                      wsem.at[par]),
                pltpu.make_async_copy(bufcc.at[par],
                                      occ_hbm.at[pl.ds(base, CH)],
                                      wsem.at[par]),
            )

        def gdescs(par):
            return (
                pltpu.make_async_copy(h_hbm.at[idxr.at[par]], bufhr.at[par],
                                      gsem.at[par]),
                pltpu.make_async_copy(h_hbm.at[idxc.at[par]], bufhc.at[par],
                                      gsem.at[par]),
                pltpu.make_async_copy(c_hbm.at[idxr.at[par]], bufcr.at[par],
                                      gsem.at[par]),
                pltpu.make_async_copy(c_hbm.at[idxc.at[par]], bufcc.at[par],
                                      gsem.at[par]),
            )

        def start_chunk(c, par):
            base = wbase + c * CH
            pltpu.sync_copy(row_hbm.at[pl.ds(base, CH)], idxr.at[par])
            pltpu.sync_copy(col_hbm.at[pl.ds(base, CH)], idxc.at[par])
            for d in gdescs(par):
                d.start()

        start_chunk(0, 0)

        def body(c, carry):
            par = lax.rem(c, 2)
            npar = 1 - par

            @pl.when(c >= 1)
            def _():
                for d in wdescs(c - 1, npar):
                    d.wait()

            @pl.when(c < NCHK - 1)
            def _():
                start_chunk(c + 1, npar)

            for d in gdescs(par):
                d.wait()
            for d in wdescs(c, par):
                d.start()
            return carry

        lax.fori_loop(0, NCHK, body, 0)
        for d in wdescs(NCHK - 1, (NCHK - 1) % 2):
            d.wait()

    return k(h, cpad, row, col)


def _sc_scatter(msg, cdw, row, zh, zc):
    E = row.shape[0]
    N, D = zh.shape
    EW = E // NW
    NCHK = EW // CH
    RT = N // NS

    @functools.partial(
        pl.kernel,
        out_type=(
            jax.ShapeDtypeStruct((NC * N, D), jnp.float32),
            jax.ShapeDtypeStruct((NC * N, CW), jnp.float32),
        ),
        mesh=_sc_mesh(),
        scratch_types=[
            pltpu.VMEM((2, CH), jnp.int32),
            pltpu.VMEM((2, CH, D), jnp.float32),
            pltpu.VMEM((2, CH, CW), jnp.float32),
            pltpu.SemaphoreType.DMA((2,)),
            pltpu.VMEM_SHARED((N, D), jnp.float32),
            pltpu.VMEM_SHARED((N, CW), jnp.float32),
        ],
        compiler_params=pltpu.CompilerParams(use_tc_tiling_on_sc=False),
    )
    def k(msg_hbm, cdw_hbm, row_hbm, zh_hbm, zc_hbm, oh_hbm, oc_hbm,
          idx, bufh, bufc, lsem, acch, accc):
        cid = lax.axis_index("c")
        sid = lax.axis_index("s")
        wid = sid * NC + cid
        wbase = wid * EW

        def ldescs(c, par):
            base = wbase + c * CH
            return (
                pltpu.make_async_copy(row_hbm.at[pl.ds(base, CH)],
                                      idx.at[par], lsem.at[par]),
                pltpu.make_async_copy(msg_hbm.at[pl.ds(base, CH)],
                                      bufh.at[par], lsem.at[par]),
                pltpu.make_async_copy(cdw_hbm.at[pl.ds(base, CH)],
                                      bufc.at[par], lsem.at[par]),
            )

        for d in ldescs(0, 0):
            d.start()
        pltpu.sync_copy(zh_hbm.at[pl.ds(sid * RT, RT)], acch.at[pl.ds(sid * RT, RT)])
        pltpu.sync_copy(zc_hbm.at[pl.ds(sid * RT, RT)], accc.at[pl.ds(sid * RT, RT)])
        plsc.subcore_barrier()

        def body(c, carry):
            par = lax.rem(c, 2)
            npar = 1 - par

            @pl.when(c < NCHK - 1)
            def _():
                for d in ldescs(c + 1, npar):
                    d.start()

            for d in ldescs(c, par):
                d.wait()
            pltpu.sync_copy(bufh.at[par], acch.at[idx.at[par]], add=True)
            pltpu.sync_copy(bufc.at[par], accc.at[idx.at[par]], add=True)
            return carry

        lax.fori_loop(0, NCHK, body, 0)
        plsc.subcore_barrier()
        pltpu.sync_copy(
            acch.at[pl.ds(sid * RT, RT)],
            oh_hbm.at[pl.ds(cid * N + sid * RT, RT)],
        )
        pltpu.sync_copy(
            accc.at[pl.ds(sid * RT, RT)],
            oc_hbm.at[pl.ds(cid * N + sid * RT, RT)],
        )

    return k(msg, cdw, row, zh, zc)


def _tc_edge(hrow, hcol, crow, ccol, Wa, Wb, wr, be1, We2, be2, Wc1, bc1, wc2):
    E, D = hrow.shape
    BE = 1600

    def body(hr_ref, hc_ref, cr_ref, cc_ref, Wa_ref, Wb_ref, wr_ref, be1_ref,
             We2_ref, be2_ref, Wc1_ref, bc1_ref, wc2_ref, om_ref, oc_ref):
        bf = jnp.bfloat16
        cd = cr_ref[...] - cc_ref[...]
        radial = jnp.sum(cd * cd, axis=-1, keepdims=True)
        t1 = _silu(
            jnp.dot(hr_ref[...].astype(bf), Wa_ref[...].astype(bf),
                    preferred_element_type=jnp.float32)
            + jnp.dot(hc_ref[...].astype(bf), Wb_ref[...].astype(bf),
                      preferred_element_type=jnp.float32)
            + radial * wr_ref[...]
            + be1_ref[...]
        )
        msg = _silu(
            jnp.dot(t1.astype(bf), We2_ref[...].astype(bf),
                    preferred_element_type=jnp.float32)
            + be2_ref[...]
        )
        t3 = _silu(
            jnp.dot(msg.astype(bf), Wc1_ref[...].astype(bf),
                    preferred_element_type=jnp.float32)
            + bc1_ref[...]
        )
        cw = jnp.sum(t3 * wc2_ref[...], axis=-1, keepdims=True)
        om_ref[...] = msg
        oc_ref[...] = cd * cw

    wspec = pl.BlockSpec((128, 128), lambda i: (0, 0))
    vspec = pl.BlockSpec((1, 128), lambda i: (0, 0))
    espec = pl.BlockSpec((BE, D), lambda i: (i, 0))
    cspec = pl.BlockSpec((BE, CW), lambda i: (i, 0))
    return pl.pallas_call(
        body,
        grid=(E // BE,),
        in_specs=[
            espec, espec, cspec, cspec,
            wspec, wspec, vspec, vspec, wspec, vspec, wspec, vspec, vspec,
        ],
        out_specs=[espec, cspec],
        out_shape=[
            jax.ShapeDtypeStruct((E, D), jnp.float32),
            jax.ShapeDtypeStruct((E, CW), jnp.float32),
        ],
        compiler_params=pltpu.CompilerParams(
            dimension_semantics=("arbitrary",)
        ),
    )(hrow, hcol, crow, ccol, Wa, Wb, wr, be1, We2, be2, Wc1, bc1, wc2)


def _tc_node(h, cpad, p0h, p1h, p0c, p1c, Wn1a, Wn1b, bn1, Wn2, bn2):
    N, D = h.shape
    BN = 2000

    def body(h_ref, cp_ref, p0h_ref, p1h_ref, p0c_ref, p1c_ref, Wa_ref,
             Wb_ref, b1_ref, W2_ref, b2_ref, oh_ref, oc_ref):
        bf = jnp.bfloat16
        hh = h_ref[...]
        agg = p0h_ref[...] + p1h_ref[...]
        t = _silu(
            jnp.dot(hh.astype(bf), Wa_ref[...].astype(bf),
                    preferred_element_type=jnp.float32)
            + jnp.dot(agg.astype(bf), Wb_ref[...].astype(bf),
                      preferred_element_type=jnp.float32)
            + b1_ref[...]
        )
        oh_ref[...] = (
            jnp.dot(t.astype(bf), W2_ref[...].astype(bf),
                    preferred_element_type=jnp.float32)
            + b2_ref[...]
            + hh
        )
        oc_ref[...] = cp_ref[...] + p0c_ref[...] + p1c_ref[...]

    wspec = pl.BlockSpec((128, 128), lambda i: (0, 0))
    vspec = pl.BlockSpec((1, 128), lambda i: (0, 0))
    nspec = pl.BlockSpec((BN, D), lambda i: (i, 0))
    cspec = pl.BlockSpec((BN, CW), lambda i: (i, 0))
    return pl.pallas_call(
        body,
        grid=(N // BN,),
        in_specs=[nspec, cspec, nspec, nspec, cspec, cspec,
                  wspec, wspec, vspec, wspec, vspec],
        out_specs=[nspec, cspec],
        out_shape=[
            jax.ShapeDtypeStruct((N, D), jnp.float32),
            jax.ShapeDtypeStruct((N, CW), jnp.float32),
        ],
        compiler_params=pltpu.CompilerParams(
            dimension_semantics=("arbitrary",)
        ),
    )(h, cpad, p0h, p1h, p0c, p1c, Wn1a, Wn1b, bn1, Wn2, bn2)


def kernel(h, coords, edge_index, We1, be1, We2, be2, Wn1, bn1, Wn2, bn2, Wc1,
           bc1, Wc2):
    N, D = h.shape
    row = edge_index[0].astype(jnp.int32)
    col = edge_index[1].astype(jnp.int32)
    cpad = jnp.pad(
        coords.astype(jnp.float32), ((0, 0), (0, CW - coords.shape[1]))
    )

    hrow, hcol, crow, ccol = _sc_gather(h, cpad, row, col)

    Wa = We1[0:D]
    Wb = We1[D:2 * D]
    wr = We1[2 * D].reshape(1, D)
    msg, cdw = _tc_edge(
        hrow, hcol, crow, ccol, Wa, Wb, wr, be1.reshape(1, D), We2,
        be2.reshape(1, D), Wc1, bc1.reshape(1, D), Wc2.reshape(1, D),
    )

    zh = jnp.zeros((N, D), jnp.float32)
    zc = jnp.zeros((N, CW), jnp.float32)
    ph, pc = _sc_scatter(msg, cdw, row, zh, zc)

    h_new, coords_new_pad = _tc_node(
        h, cpad, ph[0:N], ph[N:2 * N], pc[0:N], pc[N:2 * N],
        Wn1[0:D], Wn1[D:2 * D], bn1.reshape(1, D), Wn2, bn2.reshape(1, D),
    )
    return h_new, coords_new_pad[:, 0:coords.shape[1]]


# speedup vs baseline: 1.0955x; 1.0026x over previous
import functools

import jax
import jax.numpy as jnp
from jax import lax
from jax.experimental import pallas as pl
from jax.experimental.pallas import tpu as pltpu
from jax.experimental.pallas import tpu_sc as plsc

NC = 2
NS = 16
NW = NC * NS
CH = 80
CW = 16


def _silu(x):
    return x * (1.0 / (1.0 + jnp.exp(-x)))


def _sc_mesh():
    return plsc.VectorSubcoreMesh(
        core_axis_name="c", subcore_axis_name="s", num_cores=NC, num_subcores=NS
    )


def _sc_gather(h, cpad, row, col):
    E = row.shape[0]
    N, D = h.shape
    EW = E // NW
    NCHK = EW // CH

    @functools.partial(
        pl.kernel,
        out_type=(
            jax.ShapeDtypeStruct((E, D), jnp.float32),
            jax.ShapeDtypeStruct((E, D), jnp.float32),
            jax.ShapeDtypeStruct((E, CW), jnp.float32),
            jax.ShapeDtypeStruct((E, CW), jnp.float32),
        ),
        mesh=_sc_mesh(),
        scratch_types=[
            pltpu.VMEM((2, CH), jnp.int32),
            pltpu.VMEM((2, CH), jnp.int32),
            pltpu.VMEM((2, CH, D), jnp.float32),
            pltpu.VMEM((2, CH, D), jnp.float32),
            pltpu.VMEM((2, CH, CW), jnp.float32),
            pltpu.VMEM((2, CH, CW), jnp.float32),
            pltpu.SemaphoreType.DMA((2,)),
            pltpu.SemaphoreType.DMA((2,)),
        ],
        compiler_params=pltpu.CompilerParams(use_tc_tiling_on_sc=False),
    )
    def k(h_hbm, c_hbm, row_hbm, col_hbm, ohr_hbm, ohc_hbm, ocr_hbm, occ_hbm,
          idxr, idxc, bufhr, bufhc, bufcr, bufcc, gsem, wsem):
        wid = lax.axis_index("s") * NC + lax.axis_index("c")
        wbase = wid * EW

        def wdescs(c, par):
            base = wbase + c * CH
            return (
                pltpu.make_async_copy(bufhr.at[par],
                                      ohr_hbm.at[pl.ds(base, CH)],
                                      wsem.at[par]),
                pltpu.make_async_copy(bufhc.at[par],
                                      ohc_hbm.at[pl.ds(base, CH)],
                                      wsem.at[par]),
                pltpu.make_async_copy(bufcr.at[par],
                                      ocr_hbm.at[pl.ds(base, CH)],
                                      wsem.at[par]),
                pltpu.make_async_copy(bufcc.at[par],
                                      occ_hbm.at[pl.ds(base, CH)],
                                      wsem.at[par]),
            )

        def gdescs(par):
            return (
                pltpu.make_async_copy(h_hbm.at[idxr.at[par]], bufhr.at[par],
                                      gsem.at[par]),
                pltpu.make_async_copy(h_hbm.at[idxc.at[par]], bufhc.at[par],
                                      gsem.at[par]),
                pltpu.make_async_copy(c_hbm.at[idxr.at[par]], bufcr.at[par],
                                      gsem.at[par]),
                pltpu.make_async_copy(c_hbm.at[idxc.at[par]], bufcc.at[par],
                                      gsem.at[par]),
            )

        def start_chunk(c, par):
            base = wbase + c * CH
            pltpu.sync_copy(row_hbm.at[pl.ds(base, CH)], idxr.at[par])
            pltpu.sync_copy(col_hbm.at[pl.ds(base, CH)], idxc.at[par])
            for d in gdescs(par):
                d.start()

        start_chunk(0, 0)

        def body(c, carry):
            par = lax.rem(c, 2)
            npar = 1 - par

            @pl.when(c >= 1)
            def _():
                for d in wdescs(c - 1, npar):
                    d.wait()

            @pl.when(c < NCHK - 1)
            def _():
                start_chunk(c + 1, npar)

            for d in gdescs(par):
                d.wait()
            for d in wdescs(c, par):
                d.start()
            return carry

        lax.fori_loop(0, NCHK, body, 0)
        for d in wdescs(NCHK - 1, (NCHK - 1) % 2):
            d.wait()

    return k(h, cpad, row, col)


def _sc_scatter(msg, cdw, row, zh, zc):
    E = row.shape[0]
    N, D = zh.shape
    EW = E // NW
    NCHK = EW // CH
    RT = N // NS

    @functools.partial(
        pl.kernel,
        out_type=(
            jax.ShapeDtypeStruct((NC * N, D), jnp.float32),
            jax.ShapeDtypeStruct((NC * N, CW), jnp.float32),
        ),
        mesh=_sc_mesh(),
        scratch_types=[
            pltpu.VMEM((2, CH), jnp.int32),
            pltpu.VMEM((2, CH, D), jnp.float32),
            pltpu.VMEM((2, CH, CW), jnp.float32),
            pltpu.SemaphoreType.DMA((2,)),
            pltpu.SemaphoreType.DMA((2,)),
            pltpu.VMEM_SHARED((N, D), jnp.float32),
            pltpu.VMEM_SHARED((N, CW), jnp.float32),
        ],
        compiler_params=pltpu.CompilerParams(use_tc_tiling_on_sc=False),
    )
    def k(msg_hbm, cdw_hbm, row_hbm, zh_hbm, zc_hbm, oh_hbm, oc_hbm,
          idx, bufh, bufc, lsem, ssem, acch, accc):
        cid = lax.axis_index("c")
        sid = lax.axis_index("s")
        wid = sid * NC + cid
        wbase = wid * EW

        def ldescs(c, par):
            base = wbase + c * CH
            return (
                pltpu.make_async_copy(row_hbm.at[pl.ds(base, CH)],
                                      idx.at[par], lsem.at[par]),
                pltpu.make_async_copy(msg_hbm.at[pl.ds(base, CH)],
                                      bufh.at[par], lsem.at[par]),
                pltpu.make_async_copy(cdw_hbm.at[pl.ds(base, CH)],
                                      bufc.at[par], lsem.at[par]),
            )

        for d in ldescs(0, 0):
            d.start()
        pltpu.sync_copy(zh_hbm.at[pl.ds(sid * RT, RT)], acch.at[pl.ds(sid * RT, RT)])
        pltpu.sync_copy(zc_hbm.at[pl.ds(sid * RT, RT)], accc.at[pl.ds(sid * RT, RT)])
        plsc.subcore_barrier()

        def sdescs(par):
            return (
                pltpu.make_async_copy(bufh.at[par], acch.at[idx.at[par]],
                                      ssem.at[par]),
                pltpu.make_async_copy(bufc.at[par], accc.at[idx.at[par]],
                                      ssem.at[par]),
            )

        def body(c, carry):
            par = lax.rem(c, 2)
            npar = 1 - par

            @pl.when(c >= 1)
            def _():
                for d in sdescs(npar):
                    d.wait()

            @pl.when(c < NCHK - 1)
            def _():
                for d in ldescs(c + 1, npar):
                    d.start()

            for d in ldescs(c, par):
                d.wait()
            for d in sdescs(par):
                d.start(add=True)
            return carry

        lax.fori_loop(0, NCHK, body, 0)
        for d in sdescs((NCHK - 1) % 2):
            d.wait()
        plsc.subcore_barrier()
        pltpu.sync_copy(
            acch.at[pl.ds(sid * RT, RT)],
            oh_hbm.at[pl.ds(cid * N + sid * RT, RT)],
        )
        pltpu.sync_copy(
            accc.at[pl.ds(sid * RT, RT)],
            oc_hbm.at[pl.ds(cid * N + sid * RT, RT)],
        )

    return k(msg, cdw, row, zh, zc)


def _tc_edge(hrow, hcol, crow, ccol, Wa, Wb, wr, be1, We2, be2, Wc1, bc1, wc2):
    E, D = hrow.shape
    BE = 1600

    def body(hr_ref, hc_ref, cr_ref, cc_ref, Wa_ref, Wb_ref, wr_ref, be1_ref,
             We2_ref, be2_ref, Wc1_ref, bc1_ref, wc2_ref, om_ref, oc_ref):
        bf = jnp.bfloat16
        cd = cr_ref[...] - cc_ref[...]
        radial = jnp.sum(cd * cd, axis=-1, keepdims=True)
        t1 = _silu(
            jnp.dot(hr_ref[...].astype(bf), Wa_ref[...].astype(bf),
                    preferred_element_type=jnp.float32)
            + jnp.dot(hc_ref[...].astype(bf), Wb_ref[...].astype(bf),
                      preferred_element_type=jnp.float32)
            + radial * wr_ref[...]
            + be1_ref[...]
        )
        msg = _silu(
            jnp.dot(t1.astype(bf), We2_ref[...].astype(bf),
                    preferred_element_type=jnp.float32)
            + be2_ref[...]
        )
        t3 = _silu(
            jnp.dot(msg.astype(bf), Wc1_ref[...].astype(bf),
                    preferred_element_type=jnp.float32)
            + bc1_ref[...]
        )
        cw = jnp.sum(t3 * wc2_ref[...], axis=-1, keepdims=True)
        om_ref[...] = msg
        oc_ref[...] = cd * cw

    wspec = pl.BlockSpec((128, 128), lambda i: (0, 0))
    vspec = pl.BlockSpec((1, 128), lambda i: (0, 0))
    espec = pl.BlockSpec((BE, D), lambda i: (i, 0))
    cspec = pl.BlockSpec((BE, CW), lambda i: (i, 0))
    return pl.pallas_call(
        body,
        grid=(E // BE,),
        in_specs=[
            espec, espec, cspec, cspec,
            wspec, wspec, vspec, vspec, wspec, vspec, wspec, vspec, vspec,
        ],
        out_specs=[espec, cspec],
        out_shape=[
            jax.ShapeDtypeStruct((E, D), jnp.float32),
            jax.ShapeDtypeStruct((E, CW), jnp.float32),
        ],
        compiler_params=pltpu.CompilerParams(
            dimension_semantics=("arbitrary",)
        ),
    )(hrow, hcol, crow, ccol, Wa, Wb, wr, be1, We2, be2, Wc1, bc1, wc2)


def _tc_node(h, cpad, p0h, p1h, p0c, p1c, Wn1a, Wn1b, bn1, Wn2, bn2):
    N, D = h.shape
    BN = 2000

    def body(h_ref, cp_ref, p0h_ref, p1h_ref, p0c_ref, p1c_ref, Wa_ref,
             Wb_ref, b1_ref, W2_ref, b2_ref, oh_ref, oc_ref):
        bf = jnp.bfloat16
        hh = h_ref[...]
        agg = p0h_ref[...] + p1h_ref[...]
        t = _silu(
            jnp.dot(hh.astype(bf), Wa_ref[...].astype(bf),
                    preferred_element_type=jnp.float32)
            + jnp.dot(agg.astype(bf), Wb_ref[...].astype(bf),
                      preferred_element_type=jnp.float32)
            + b1_ref[...]
        )
        oh_ref[...] = (
            jnp.dot(t.astype(bf), W2_ref[...].astype(bf),
                    preferred_element_type=jnp.float32)
            + b2_ref[...]
            + hh
        )
        oc_ref[...] = cp_ref[...] + p0c_ref[...] + p1c_ref[...]

    wspec = pl.BlockSpec((128, 128), lambda i: (0, 0))
    vspec = pl.BlockSpec((1, 128), lambda i: (0, 0))
    nspec = pl.BlockSpec((BN, D), lambda i: (i, 0))
    cspec = pl.BlockSpec((BN, CW), lambda i: (i, 0))
    return pl.pallas_call(
        body,
        grid=(N // BN,),
        in_specs=[nspec, cspec, nspec, nspec, cspec, cspec,
                  wspec, wspec, vspec, wspec, vspec],
        out_specs=[nspec, cspec],
        out_shape=[
            jax.ShapeDtypeStruct((N, D), jnp.float32),
            jax.ShapeDtypeStruct((N, CW), jnp.float32),
        ],
        compiler_params=pltpu.CompilerParams(
            dimension_semantics=("arbitrary",)
        ),
    )(h, cpad, p0h, p1h, p0c, p1c, Wn1a, Wn1b, bn1, Wn2, bn2)


def kernel(h, coords, edge_index, We1, be1, We2, be2, Wn1, bn1, Wn2, bn2, Wc1,
           bc1, Wc2):
    N, D = h.shape
    row = edge_index[0].astype(jnp.int32)
    col = edge_index[1].astype(jnp.int32)
    cpad = jnp.pad(
        coords.astype(jnp.float32), ((0, 0), (0, CW - coords.shape[1]))
    )

    hrow, hcol, crow, ccol = _sc_gather(h, cpad, row, col)

    Wa = We1[0:D]
    Wb = We1[D:2 * D]
    wr = We1[2 * D].reshape(1, D)
    msg, cdw = _tc_edge(
        hrow, hcol, crow, ccol, Wa, Wb, wr, be1.reshape(1, D), We2,
        be2.reshape(1, D), Wc1, bc1.reshape(1, D), Wc2.reshape(1, D),
    )

    zh = jnp.zeros((N, D), jnp.float32)
    zc = jnp.zeros((N, CW), jnp.float32)
    ph, pc = _sc_scatter(msg, cdw, row, zh, zc)

    h_new, coords_new_pad = _tc_node(
        h, cpad, ph[0:N], ph[N:2 * N], pc[0:N], pc[N:2 * N],
        Wn1[0:D], Wn1[D:2 * D], bn1.reshape(1, D), Wn2, bn2.reshape(1, D),
    )
    return h_new, coords_new_pad[:, 0:coords.shape[1]]
